# baseline (speedup 1.0000x reference)
import jax
import jax.numpy as jnp
from jax.experimental import pallas as pl
from jax.experimental.pallas import tpu as pltpu

B, C, H, W = 4, 96, 224, 224
R = B * C
QCNT = C // 4 + 1


def _body(b1v_ref, b2v_ref, b1s_ref, b2s_ref, x0_ref, x1_ref,
          o1_ref, o2_ref, cnt_vmem, cnt_smem, copy_sem, m_sem):
    a1 = jnp.abs(b1v_ref[...])
    a2 = jnp.abs(b2v_ref[...])

    def step(j, carry):
        c1, c2 = carry
        s1 = jnp.abs(b1s_ref[j, 0])
        s2 = jnp.abs(b2s_ref[j, 0])
        return (c1 + (s1 <= a1).astype(jnp.int32),
                c2 + (s2 <= a2).astype(jnp.int32))

    z = jnp.zeros((C, 1), jnp.int32)
    c1, c2 = jax.lax.fori_loop(0, C, step, (z, z))
    cnt_vmem[:, 0:1] = c1
    cnt_vmem[:, 1:2] = c2
    pltpu.make_async_copy(cnt_vmem, cnt_smem, m_sem).start()
    pltpu.make_async_copy(cnt_vmem, cnt_smem, m_sem).wait()

    def issue(r, _):
        ch = r % C
        m1 = cnt_smem[ch, 0] >= QCNT
        m2 = cnt_smem[ch, 1] >= QCNT

        @pl.when(m1)
        def _():
            pltpu.make_async_copy(x0_ref.at[r], o1_ref.at[r], copy_sem).start()

        @pl.when(jnp.logical_not(m1))
        def _():
            pltpu.make_async_copy(x1_ref.at[r], o1_ref.at[r], copy_sem).start()

        @pl.when(m2)
        def _():
            pltpu.make_async_copy(x1_ref.at[r], o2_ref.at[r], copy_sem).start()

        @pl.when(jnp.logical_not(m2))
        def _():
            pltpu.make_async_copy(x0_ref.at[r], o2_ref.at[r], copy_sem).start()

        return 0

    jax.lax.fori_loop(0, R, issue, 0)

    def drain(r, _):
        pltpu.make_async_copy(x0_ref.at[r], o1_ref.at[r], copy_sem).wait()
        pltpu.make_async_copy(x0_ref.at[r], o2_ref.at[r], copy_sem).wait()
        return 0

    jax.lax.fori_loop(0, R, drain, 0)


def kernel(x0, x1, bn1_weight, bn2_weight, bn_threshold):
    del bn_threshold
    x0r = x0.reshape(R, H, W)
    x1r = x1.reshape(R, H, W)
    b1 = bn1_weight.reshape(C, 1)
    b2 = bn2_weight.reshape(C, 1)
    out1, out2 = pl.pallas_call(
        _body,
        in_specs=[
            pl.BlockSpec(memory_space=pltpu.VMEM),
            pl.BlockSpec(memory_space=pltpu.VMEM),
            pl.BlockSpec(memory_space=pltpu.SMEM),
            pl.BlockSpec(memory_space=pltpu.SMEM),
            pl.BlockSpec(memory_space=pl.ANY),
            pl.BlockSpec(memory_space=pl.ANY),
        ],
        out_specs=[
            pl.BlockSpec(memory_space=pl.ANY),
            pl.BlockSpec(memory_space=pl.ANY),
        ],
        out_shape=[
            jax.ShapeDtypeStruct((R, H, W), jnp.float32),
            jax.ShapeDtypeStruct((R, H, W), jnp.float32),
        ],
        scratch_shapes=[
            pltpu.VMEM((C, 2), jnp.int32),
            pltpu.SMEM((C, 2), jnp.int32),
            pltpu.SemaphoreType.DMA,
            pltpu.SemaphoreType.DMA,
        ],
    )(b1, b2, b1, b2, x0r, x1r)
    return (out1.reshape(B, C, H, W), out2.reshape(B, C, H, W))


# speedup vs baseline: 47.1909x; 47.1909x over previous
import jax
import jax.numpy as jnp
from jax.experimental import pallas as pl
from jax.experimental.pallas import tpu as pltpu

B, C, H, W = 4, 96, 224, 224
R = B * C
RB = 16
GRID = R // RB
CB_COUNT = C // RB
QCNT = C // 4 + 1


def _body(x0_ref, x1_ref, b1v_ref, b2v_ref, b1s_ref, b2s_ref,
          o1_ref, o2_ref, m1_scr, m2_scr):
    i = pl.program_id(0)

    @pl.when(i < CB_COUNT)
    def _compute_masks():
        a1 = jnp.abs(b1v_ref[...])
        a2 = jnp.abs(b2v_ref[...])

        def step(j, carry):
            c1, c2 = carry
            s1 = jnp.abs(b1s_ref[j, 0])
            s2 = jnp.abs(b2s_ref[j, 0])
            c1 = c1 + (s1 <= a1).astype(jnp.int32)
            c2 = c2 + (s2 <= a2).astype(jnp.int32)
            return c1, c2

        z = jnp.zeros((RB, 1), jnp.int32)
        c1, c2 = jax.lax.fori_loop(0, C, step, (z, z))
        m1_scr[pl.ds(i * RB, RB), :] = c1
        m2_scr[pl.ds(i * RB, RB), :] = c2

    cb = (i % CB_COUNT) * RB
    m1 = jnp.reshape(m1_scr[pl.ds(cb, RB), :] >= QCNT, (RB, 1, 1))
    m2 = jnp.reshape(m2_scr[pl.ds(cb, RB), :] >= QCNT, (RB, 1, 1))
    x0 = x0_ref[...]
    x1 = x1_ref[...]
    o1_ref[...] = jnp.where(m1, x0, x1)
    o2_ref[...] = jnp.where(m2, x1, x0)


def kernel(x0, x1, bn1_weight, bn2_weight, bn_threshold):
    del bn_threshold
    x0r = x0.reshape(R, H, W)
    x1r = x1.reshape(R, H, W)
    b1 = bn1_weight.reshape(C, 1)
    b2 = bn2_weight.reshape(C, 1)
    out1, out2 = pl.pallas_call(
        _body,
        grid=(GRID,),
        in_specs=[
            pl.BlockSpec((RB, H, W), lambda i: (i, 0, 0)),
            pl.BlockSpec((RB, H, W), lambda i: (i, 0, 0)),
            pl.BlockSpec((RB, 1), lambda i: (i % CB_COUNT, 0)),
            pl.BlockSpec((RB, 1), lambda i: (i % CB_COUNT, 0)),
            pl.BlockSpec(memory_space=pltpu.SMEM),
            pl.BlockSpec(memory_space=pltpu.SMEM),
        ],
        out_specs=[
            pl.BlockSpec((RB, H, W), lambda i: (i, 0, 0)),
            pl.BlockSpec((RB, H, W), lambda i: (i, 0, 0)),
        ],
        out_shape=[
            jax.ShapeDtypeStruct((R, H, W), jnp.float32),
            jax.ShapeDtypeStruct((R, H, W), jnp.float32),
        ],
        scratch_shapes=[
            pltpu.VMEM((C, 1), jnp.int32),
            pltpu.VMEM((C, 1), jnp.int32),
        ],
    )(x0r, x1r, b1, b2, b1, b2)
    return (out1.reshape(B, C, H, W), out2.reshape(B, C, H, W))


# speedup vs baseline: 47.5774x; 1.0082x over previous
import jax
import jax.numpy as jnp
from jax.experimental import pallas as pl
from jax.experimental.pallas import tpu as pltpu

B, C, H, W = 4, 96, 224, 224
R = B * C
RB = 24
GRID = R // RB
CB_COUNT = C // RB
QCNT = C // 4 + 1


def _body(x0_ref, x1_ref, b1v_ref, b2v_ref, b1s_ref, b2s_ref,
          o1_ref, o2_ref, m1_scr, m2_scr):
    i = pl.program_id(0)

    @pl.when(i < CB_COUNT)
    def _compute_masks():
        a1 = jnp.abs(b1v_ref[...])
        a2 = jnp.abs(b2v_ref[...])

        def step(j, carry):
            c1, c2 = carry
            s1 = jnp.abs(b1s_ref[j, 0])
            s2 = jnp.abs(b2s_ref[j, 0])
            c1 = c1 + (s1 <= a1).astype(jnp.int32)
            c2 = c2 + (s2 <= a2).astype(jnp.int32)
            return c1, c2

        z = jnp.zeros((RB, 1), jnp.int32)
        c1, c2 = jax.lax.fori_loop(0, C, step, (z, z))
        m1_scr[pl.ds(i * RB, RB), :] = c1
        m2_scr[pl.ds(i * RB, RB), :] = c2

    cb = (i % CB_COUNT) * RB
    m1 = jnp.reshape(m1_scr[pl.ds(cb, RB), :] >= QCNT, (RB, 1, 1))
    m2 = jnp.reshape(m2_scr[pl.ds(cb, RB), :] >= QCNT, (RB, 1, 1))
    x0 = x0_ref[...]
    x1 = x1_ref[...]
    o1_ref[...] = jnp.where(m1, x0, x1)
    o2_ref[...] = jnp.where(m2, x1, x0)


def kernel(x0, x1, bn1_weight, bn2_weight, bn_threshold):
    del bn_threshold
    x0r = x0.reshape(R, H, W)
    x1r = x1.reshape(R, H, W)
    b1 = bn1_weight.reshape(C, 1)
    b2 = bn2_weight.reshape(C, 1)
    out1, out2 = pl.pallas_call(
        _body,
        grid=(GRID,),
        in_specs=[
            pl.BlockSpec((RB, H, W), lambda i: (i, 0, 0)),
            pl.BlockSpec((RB, H, W), lambda i: (i, 0, 0)),
            pl.BlockSpec((RB, 1), lambda i: (i % CB_COUNT, 0)),
            pl.BlockSpec((RB, 1), lambda i: (i % CB_COUNT, 0)),
            pl.BlockSpec(memory_space=pltpu.SMEM),
            pl.BlockSpec(memory_space=pltpu.SMEM),
        ],
        out_specs=[
            pl.BlockSpec((RB, H, W), lambda i: (i, 0, 0)),
            pl.BlockSpec((RB, H, W), lambda i: (i, 0, 0)),
        ],
        out_shape=[
            jax.ShapeDtypeStruct((R, H, W), jnp.float32),
            jax.ShapeDtypeStruct((R, H, W), jnp.float32),
        ],
        scratch_shapes=[
            pltpu.VMEM((C, 1), jnp.int32),
            pltpu.VMEM((C, 1), jnp.int32),
        ],
    )(x0r, x1r, b1, b2, b1, b2)
    return (out1.reshape(B, C, H, W), out2.reshape(B, C, H, W))
